# baseline (speedup 1.0000x reference)
import functools

import jax
import jax.numpy as jnp
from jax import lax
from jax.experimental import pallas as pl
from jax.experimental.pallas import tpu as pltpu
from jax.experimental.pallas import tpu_sc as plsc

N = 10000
E = 320000
D = 128

NC = 2
NS = 16
NW = NC * NS
L = 16

RPT = 320
NPAD = NW * RPT
CH = 2000
NCHK = E // CH
G = 128
FG = D // L


def _segmax_body(x_hbm, dst_hbm, src_hbm, out_hbm,
                 dst_v, src_v, pldst_v, psrc_v, rows_v, agg_v, sem):
    w = lax.axis_index("s") * NC + lax.axis_index("c")
    lo = w * RPT
    iota = lax.iota(jnp.int32, L)
    ninf = jnp.full((L,), -jnp.inf, jnp.float32)

    def _init(i, _):
        r = i // FG
        f = i - r * FG
        agg_v[r, pl.ds(f * L, L)] = ninf
        return 0
    lax.fori_loop(0, (RPT + 1) * FG, _init, 0)

    def _chunk(c, _):
        pltpu.sync_copy(dst_hbm.at[pl.ds(c * CH, CH)], dst_v)
        pltpu.sync_copy(src_hbm.at[pl.ds(c * CH, CH)], src_v)

        def _filt(i, cnt):
            d = dst_v[pl.ds(i * L, L)]
            s = src_v[pl.ds(i * L, L)]
            ld = d - lo
            m = (ld >= 0) & (ld < RPT)
            mi = m.astype(jnp.int32)
            pos = cnt + jnp.cumsum(mi) - 1
            plsc.store_scatter(pldst_v, [pos], ld, mask=m)
            plsc.store_scatter(psrc_v, [pos], s, mask=m)
            return cnt + jnp.sum(mi)
        cnt = lax.fori_loop(0, CH // L, _filt, 0)

        for j in range(G // L):
            tidx = cnt + j * L + iota
            plsc.store_scatter(pldst_v, [tidx], jnp.full((L,), RPT, jnp.int32))
            plsc.store_scatter(psrc_v, [tidx], jnp.zeros((L,), jnp.int32))

        ngroups = (cnt + G - 1) // G

        def _group(g, _):
            cps = []
            for j in range(G // 16):
                cps.append(pltpu.async_copy(
                    x_hbm.at[psrc_v.at[pl.ds(g * G + j * 16, 16)]],
                    rows_v.at[pl.ds(j * 16, 16)], sem))
            for cp in cps:
                cp.wait()

            def _edge(e, _):
                evec = jnp.full((L,), e, jnp.int32)
                dvec = plsc.load_gather(pldst_v, [jnp.full((L,), g * G, jnp.int32) + evec])
                if True:
                    plsc.store_scatter(agg_v, [dvec, iota], jnp.full((L,), 0.0, jnp.float32))
                else:
                    for f in range(FG):
                        col = iota + f * L
                        old = plsc.load_gather(agg_v, [dvec, col])
                        val = plsc.load_gather(rows_v, [evec, col])
                        plsc.store_scatter(agg_v, [dvec, col], jnp.maximum(old, val))
                return 0
            return 0
        lax.fori_loop(0, ngroups, _group, 0)
        return 0
    lax.fori_loop(0, NCHK, _chunk, 0)

    pltpu.sync_copy(agg_v.at[pl.ds(0, RPT)], out_hbm.at[pl.ds(lo, RPT)])


_segmax = functools.partial(
    pl.kernel,
    out_type=jax.ShapeDtypeStruct((NPAD, D), jnp.float32),
    mesh=plsc.VectorSubcoreMesh(core_axis_name="c", subcore_axis_name="s"),
    scratch_types=[
        pltpu.VMEM((CH,), jnp.int32),
        pltpu.VMEM((CH,), jnp.int32),
        pltpu.VMEM((CH + G,), jnp.int32),
        pltpu.VMEM((CH + G,), jnp.int32),
        pltpu.VMEM((G, D), jnp.float32),
        pltpu.VMEM((RPT + 1, D), jnp.float32),
        pltpu.SemaphoreType.DMA,
    ],
    compiler_params=pltpu.CompilerParams(
        needs_layout_passes=False, use_tc_tiling_on_sc=False),
)(_segmax_body)


def _mm_body(agg_ref, x_ref, wrel_ref, wroot_ref, b_ref, o_ref):
    agg = agg_ref[...]
    agg = jnp.where(jnp.isfinite(agg), agg, 0.0)
    o_ref[...] = (
        lax.dot_general(agg, wrel_ref[...], (((1,), (1,)), ((), ())),
                        preferred_element_type=jnp.float32)
        + lax.dot_general(x_ref[...], wroot_ref[...], (((1,), (1,)), ((), ())),
                          preferred_element_type=jnp.float32)
        + b_ref[...]
    )


def _layer_mm(agg, x, W_rel, b_rel, W_root):
    BR = 1000
    return pl.pallas_call(
        _mm_body,
        grid=(N // BR,),
        in_specs=[
            pl.BlockSpec((BR, D), lambda i: (i, 0)),
            pl.BlockSpec((BR, D), lambda i: (i, 0)),
            pl.BlockSpec((D, D), lambda i: (0, 0)),
            pl.BlockSpec((D, D), lambda i: (0, 0)),
            pl.BlockSpec((1, D), lambda i: (0, 0)),
        ],
        out_specs=pl.BlockSpec((BR, D), lambda i: (i, 0)),
        out_shape=jax.ShapeDtypeStruct((N, D), jnp.float32),
    )(agg, x, W_rel, W_root, b_rel.reshape(1, D))


def kernel(x, edge_index, W_rel1, b_rel1, W_root1, W_rel2, b_rel2, W_root2):
    src = edge_index[0]
    dst = edge_index[1]
    agg1 = _segmax(x, dst, src)
    h1 = _layer_mm(agg1[:N], x, W_rel1, b_rel1, W_root1)
    agg2 = _segmax(h1, dst, src)
    h2 = _layer_mm(agg2[:N], h1, W_rel2, b_rel2, W_root2)
    return h2


# speedup vs baseline: 9.2252x; 9.2252x over previous
import functools

import jax
import jax.numpy as jnp
from jax import lax
from jax.experimental import pallas as pl
from jax.experimental.pallas import tpu as pltpu
from jax.experimental.pallas import tpu_sc as plsc

N = 10000
E = 320000
D = 128

NC = 2
NS = 16
L = 16
DH = D // NC

RPT = 640
NPAD = NS * RPT
CH = 4000
NCHK = E // CH
G = 256
FG = DH // L


def _segmax_body(x_hbm, dst_hbm, src_hbm, out_hbm,
                 dst_v, src_v, pldst_v, psrc_v, rows_v, agg_v, xs_sh, sem):
    c = lax.axis_index("c")
    s = lax.axis_index("s")
    lo = s * RPT
    iota = lax.iota(jnp.int32, L)
    ninf = jnp.full((L,), -jnp.inf, jnp.float32)

    @pl.when(s == 0)
    def _stage():
        pltpu.sync_copy(x_hbm.at[c], xs_sh)
    plsc.subcore_barrier()

    def _init(i, _):
        r = i // FG
        f = i - r * FG
        agg_v[r, pl.ds(f * L, L)] = ninf
        return 0
    lax.fori_loop(0, (RPT + 1) * FG, _init, 0)

    def _chunk(ci, _):
        pltpu.sync_copy(dst_hbm.at[pl.ds(ci * CH, CH)], dst_v)
        pltpu.sync_copy(src_hbm.at[pl.ds(ci * CH, CH)], src_v)

        def _filt(i, cnt):
            d = dst_v[pl.ds(i * L, L)]
            sv = src_v[pl.ds(i * L, L)]
            ld = d - lo
            m = (ld >= 0) & (ld < RPT)
            mi = m.astype(jnp.int32)
            pos = cnt + jnp.cumsum(mi) - 1
            plsc.store_scatter(pldst_v, [pos], ld, mask=m)
            plsc.store_scatter(psrc_v, [pos], sv, mask=m)
            return cnt + jnp.sum(mi)
        cnt = lax.fori_loop(0, CH // L, _filt, 0)

        for j in range(G // L):
            tidx = cnt + j * L + iota
            plsc.store_scatter(pldst_v, [tidx], jnp.full((L,), RPT, jnp.int32))
            plsc.store_scatter(psrc_v, [tidx], jnp.zeros((L,), jnp.int32))

        ngroups = (cnt + G - 1) // G

        def _group(g, _):
            cp = pltpu.async_copy(
                xs_sh.at[psrc_v.at[pl.ds(g * G, G)]], rows_v, sem)
            cp.wait()

            def _edge(e, _):
                evec = jnp.full((L,), e, jnp.int32)
                dvec = plsc.load_gather(
                    pldst_v, [jnp.full((L,), g * G, jnp.int32) + evec])
                for f in range(FG):
                    col = iota + f * L
                    old = plsc.load_gather(agg_v, [dvec, col])
                    val = plsc.load_gather(rows_v, [evec, col])
                    plsc.store_scatter(agg_v, [dvec, col], jnp.maximum(old, val))
                return 0
            lax.fori_loop(0, G, _edge, 0)
            return 0
        lax.fori_loop(0, ngroups, _group, 0)
        return 0
    lax.fori_loop(0, NCHK, _chunk, 0)

    pltpu.sync_copy(agg_v.at[pl.ds(0, RPT)], out_hbm.at[c, pl.ds(lo, RPT)])


_segmax = functools.partial(
    pl.kernel,
    out_type=jax.ShapeDtypeStruct((NC, NPAD, DH), jnp.float32),
    mesh=plsc.VectorSubcoreMesh(core_axis_name="c", subcore_axis_name="s"),
    scratch_types=[
        pltpu.VMEM((CH,), jnp.int32),
        pltpu.VMEM((CH,), jnp.int32),
        pltpu.VMEM((CH + G,), jnp.int32),
        pltpu.VMEM((CH + G,), jnp.int32),
        pltpu.VMEM((G, DH), jnp.float32),
        pltpu.VMEM((RPT + 1, DH), jnp.float32),
        pltpu.VMEM_SHARED((N, DH), jnp.float32),
        pltpu.SemaphoreType.DMA,
    ],
    compiler_params=pltpu.CompilerParams(
        needs_layout_passes=False, use_tc_tiling_on_sc=False),
)(_segmax_body)


def _mm_body(agg_ref, x_ref, wrel_ref, wroot_ref, b_ref, o_ref):
    agg = jnp.concatenate([agg_ref[0], agg_ref[1]], axis=1)
    agg = jnp.where(jnp.isfinite(agg), agg, 0.0)
    x = jnp.concatenate([x_ref[0], x_ref[1]], axis=1)
    h = (
        lax.dot_general(agg, wrel_ref[...], (((1,), (1,)), ((), ())),
                        preferred_element_type=jnp.float32)
        + lax.dot_general(x, wroot_ref[...], (((1,), (1,)), ((), ())),
                          preferred_element_type=jnp.float32)
        + b_ref[...]
    )
    o_ref[0] = h[:, :DH]
    o_ref[1] = h[:, DH:]


def _mm_body_final(agg_ref, x_ref, wrel_ref, wroot_ref, b_ref, o_ref):
    agg = jnp.concatenate([agg_ref[0], agg_ref[1]], axis=1)
    agg = jnp.where(jnp.isfinite(agg), agg, 0.0)
    x = jnp.concatenate([x_ref[0], x_ref[1]], axis=1)
    o_ref[...] = (
        lax.dot_general(agg, wrel_ref[...], (((1,), (1,)), ((), ())),
                        preferred_element_type=jnp.float32)
        + lax.dot_general(x, wroot_ref[...], (((1,), (1,)), ((), ())),
                          preferred_element_type=jnp.float32)
        + b_ref[...]
    )


BR = 1000


def _layer_mm(agg_t, x_t, W_rel, b_rel, W_root, split_out):
    split_spec = pl.BlockSpec((NC, BR, DH), lambda i: (0, i, 0))
    if split_out:
        body, out_shape, out_spec = (
            _mm_body, jax.ShapeDtypeStruct((NC, N, DH), jnp.float32), split_spec)
    else:
        body, out_shape, out_spec = (
            _mm_body_final, jax.ShapeDtypeStruct((N, D), jnp.float32),
            pl.BlockSpec((BR, D), lambda i: (i, 0)))
    return pl.pallas_call(
        body,
        grid=(N // BR,),
        in_specs=[
            split_spec,
            split_spec,
            pl.BlockSpec((D, D), lambda i: (0, 0)),
            pl.BlockSpec((D, D), lambda i: (0, 0)),
            pl.BlockSpec((1, D), lambda i: (0, 0)),
        ],
        out_specs=out_spec,
        out_shape=out_shape,
    )(agg_t, x_t, W_rel, W_root, b_rel.reshape(1, D))


def kernel(x, edge_index, W_rel1, b_rel1, W_root1, W_rel2, b_rel2, W_root2):
    src = edge_index[0]
    dst = edge_index[1]
    x_t = jnp.transpose(x.reshape(N, NC, DH), (1, 0, 2))
    agg1_t = _segmax(x_t, dst, src)[:, :N, :]
    h1_t = _layer_mm(agg1_t, x_t, W_rel1, b_rel1, W_root1, split_out=True)
    agg2_t = _segmax(h1_t, dst, src)[:, :N, :]
    h2 = _layer_mm(agg2_t, h1_t, W_rel2, b_rel2, W_root2, split_out=False)
    return h2
